# baseline (speedup 1.0000x reference)
import jax
import jax.numpy as jnp
from jax.experimental import pallas as pl

_TEMPERATURE = 0.07
_NEG_INF = -1e30
_NUM_CLASSES = 16


def _loss_kernel(tok_ref, lab_ref, out_ref):
    scale = jnp.float32(1.0 / _TEMPERATURE) ** 0.5
    x = tok_ref[:, :] * scale
    lab = lab_ref[:, :]
    B = lab.shape[0]
    D = 64

    cls = jax.lax.broadcasted_iota(jnp.int32, (B, _NUM_CLASSES), 1)
    onehot = jnp.where(lab == cls, jnp.float32(1.0), jnp.float32(0.0))
    counts = jnp.sum(onehot, axis=0, keepdims=True)
    pos_count = jnp.sum(onehot * counts, axis=1, keepdims=True) - 1.0
    valid = pos_count > 0.0
    inv_cnt = 1.0 / jnp.maximum(pos_count, 1.0)

    row = jax.lax.broadcasted_iota(jnp.int32, (B, B), 0)
    col = jax.lax.broadcasted_iota(jnp.int32, (B, B), 1)
    eye = row == col

    M = x.shape[1] // D
    total = jnp.float32(0.0)
    for j in range(M):
        tj = x[:, j * D:(j + 1) * D]
        S = jax.lax.dot_general(
            tj, tj, (((1,), (1,)), ((), ())),
            preferred_element_type=jnp.float32)
        Sm = jnp.where(eye, jnp.float32(_NEG_INF), S)
        m = jnp.max(Sm, axis=1, keepdims=True)
        lse = m + jnp.log(jnp.sum(jnp.exp(Sm - m), axis=1, keepdims=True))
        csum = jax.lax.dot_general(
            onehot, tj, (((0,), (0,)), ((), ())),
            preferred_element_type=jnp.float32)
        g = jax.lax.dot_general(
            onehot, csum, (((1,), (0,)), ((), ())),
            preferred_element_type=jnp.float32)
        pos_dot = jnp.sum(tj * g, axis=1, keepdims=True)
        self_sq = jnp.sum(tj * tj, axis=1, keepdims=True)
        pos_sum = (pos_dot - self_sq) - pos_count * lse
        total = total + jnp.sum(jnp.where(valid, pos_sum * inv_cnt, 0.0))

    nvalid = jnp.sum(jnp.where(valid, jnp.float32(1.0), jnp.float32(0.0)))
    out_ref[:, :] = (total / (-jnp.float32(M) * nvalid)).reshape(1, 1)


def kernel(tokens, labels):
    if tokens.ndim == 2:
        tokens = tokens[:, None, :]
    B, M, D = tokens.shape
    x = tokens.astype(jnp.float32).reshape(B, M * D)
    lab = labels.astype(jnp.int32).reshape(B, 1)
    out = pl.pallas_call(
        _loss_kernel,
        out_shape=jax.ShapeDtypeStruct((1, 1), jnp.float32),
    )(x, lab)
    return out[0, 0]


# speedup vs baseline: 1.4310x; 1.4310x over previous
import jax
import jax.numpy as jnp
from jax.experimental import pallas as pl

_TEMPERATURE = 0.07
_NEG_INF = -1e30
_NUM_CLASSES = 16


def _loss_kernel(tok_ref, lc_ref, lr_ref, out_ref):
    lc = lc_ref[:, :]
    lr = lr_ref[:, :]
    B = lc.shape[0]
    C = _NUM_CLASSES

    cls_col = jax.lax.broadcasted_iota(jnp.int32, (B, C), 1)
    onehot = jnp.where(lc == cls_col, jnp.float32(1.0), jnp.float32(0.0))
    cls_row = jax.lax.broadcasted_iota(jnp.int32, (C, B), 0)
    onehot_t = jnp.where(cls_row == lr, jnp.float32(1.0), jnp.float32(0.0))

    cnt = jnp.sum(onehot_t, axis=1, keepdims=True)
    valid_cls = cnt > 1.0
    inv_cm1 = 1.0 / jnp.maximum(cnt - 1.0, 1.0)
    valid_f = jnp.where(valid_cls, jnp.float32(1.0), jnp.float32(0.0))
    nvalid = jnp.sum(cnt * valid_f)
    validmask = jax.lax.dot_general(
        onehot, valid_f, (((1,), (0,)), ((), ())),
        preferred_element_type=jnp.float32)

    row = jax.lax.broadcasted_iota(jnp.int32, (B, B), 0)
    col = jax.lax.broadcasted_iota(jnp.int32, (B, B), 1)
    eye = row == col

    M = tok_ref.shape[0]
    scale = jnp.float32(1.0 / _TEMPERATURE) ** 0.5
    total = jnp.float32(0.0)
    for j in range(M):
        tj = tok_ref[j] * scale
        S = jax.lax.dot_general(
            tj, tj, (((1,), (1,)), ((), ())),
            preferred_element_type=jnp.float32)
        Sm = jnp.where(eye, jnp.float32(_NEG_INF), S)
        m = jnp.max(Sm, axis=1, keepdims=True)
        lse = m + jnp.log(jnp.sum(jnp.exp(Sm - m), axis=1, keepdims=True))
        csum = jax.lax.dot_general(
            onehot_t, tj, (((1,), (0,)), ((), ())),
            preferred_element_type=jnp.float32)
        sqsum_cls = jax.lax.dot_general(
            onehot_t, tj * tj, (((1,), (0,)), ((), ())),
            preferred_element_type=jnp.float32)
        sq_cls = jnp.sum(sqsum_cls, axis=1, keepdims=True)
        csq = jnp.sum(csum * csum, axis=1, keepdims=True)
        pos_term = jnp.sum(valid_f * (csq - sq_cls) * inv_cm1)
        lse_term = jnp.sum(validmask * lse)
        total = total + pos_term - lse_term

    out_ref[:, :] = (total / (-jnp.float32(M) * nvalid)).reshape(1, 1)


def kernel(tokens, labels):
    if tokens.ndim == 2:
        tokens = tokens[:, None, :]
    tokens = jnp.transpose(tokens, (1, 0, 2)).astype(jnp.float32)
    labels = labels.astype(jnp.int32)
    B = tokens.shape[1]
    lc = labels.reshape(B, 1)
    lr = labels.reshape(1, B)
    out = pl.pallas_call(
        _loss_kernel,
        out_shape=jax.ShapeDtypeStruct((1, 1), jnp.float32),
    )(tokens, lc, lr)
    return out[0, 0]


# speedup vs baseline: 1.4680x; 1.0259x over previous
import jax
import jax.numpy as jnp
from jax.experimental import pallas as pl

_TEMPERATURE = 0.07
_NEG_INF = -1e30
_NUM_CLASSES = 16


def _loss_kernel(tok_ref, lc_ref, lr_ref, out_ref):
    lc = lc_ref[:, :]
    lr = lr_ref[:, :]
    B = lc.shape[0]
    C = _NUM_CLASSES

    cls_col = jax.lax.broadcasted_iota(jnp.int32, (B, C), 1)
    onehot = jnp.where(lc == cls_col, jnp.float32(1.0), jnp.float32(0.0))
    cls_row = jax.lax.broadcasted_iota(jnp.int32, (C, B), 0)
    onehot_t = jnp.where(cls_row == lr, jnp.float32(1.0), jnp.float32(0.0))

    cnt = jnp.sum(onehot_t, axis=1, keepdims=True)
    valid_cls = cnt > 1.0
    inv_cm1 = 1.0 / jnp.maximum(cnt - 1.0, 1.0)
    valid_f = jnp.where(valid_cls, jnp.float32(1.0), jnp.float32(0.0))
    nvalid = jnp.sum(cnt * valid_f)
    validmask = jax.lax.dot_general(
        onehot, valid_f, (((1,), (0,)), ((), ())),
        preferred_element_type=jnp.float32)

    row = jax.lax.broadcasted_iota(jnp.int32, (B, B), 0)
    col = jax.lax.broadcasted_iota(jnp.int32, (B, B), 1)
    eye = row == col

    M = tok_ref.shape[0]
    scale = jnp.float32(1.4426950408889634 / _TEMPERATURE) ** 0.5
    total = jnp.float32(0.0)
    for j in range(M):
        tj = tok_ref[j] * scale
        tb = tj.astype(jnp.bfloat16)
        S = jax.lax.dot_general(
            tb, tb, (((1,), (1,)), ((), ())),
            preferred_element_type=jnp.float32)
        Sm = jnp.where(eye, jnp.float32(_NEG_INF), S)
        m = jnp.max(Sm, axis=1, keepdims=True)
        lse = m + jnp.log2(jnp.sum(jnp.exp2(Sm - m), axis=1, keepdims=True))
        csum = jax.lax.dot_general(
            onehot_t, tj, (((1,), (0,)), ((), ())),
            preferred_element_type=jnp.float32)
        sqsum_cls = jax.lax.dot_general(
            onehot_t, tj * tj, (((1,), (0,)), ((), ())),
            preferred_element_type=jnp.float32)
        sq_cls = jnp.sum(sqsum_cls, axis=1, keepdims=True)
        csq = jnp.sum(csum * csum, axis=1, keepdims=True)
        pos_term = jnp.sum(valid_f * (csq - sq_cls) * inv_cm1)
        lse_term = jnp.sum(validmask * lse)
        total = total + pos_term - lse_term

    total = total * jnp.float32(0.6931471805599453)
    out_ref[:, :] = (total / (-jnp.float32(M) * nvalid)).reshape(1, 1)


def kernel(tokens, labels):
    if tokens.ndim == 2:
        tokens = tokens[:, None, :]
    tokens = jnp.transpose(tokens, (1, 0, 2)).astype(jnp.float32)
    labels = labels.astype(jnp.int32)
    B = tokens.shape[1]
    lc = labels.reshape(B, 1)
    lr = labels.reshape(1, B)
    out = pl.pallas_call(
        _loss_kernel,
        out_shape=jax.ShapeDtypeStruct((1, 1), jnp.float32),
    )(tokens, lc, lr)
    return out[0, 0]
